# baseline (speedup 1.0000x reference)
import functools
import math

import jax
import jax.numpy as jnp
import numpy as np
from jax import lax
from jax.experimental import pallas as pl
from jax.experimental.pallas import tpu as pltpu
from jax.experimental.pallas import tpu_sc as plsc

_NUM_CORES = 2
_NUM_SUBCORES = 16
_NW = _NUM_CORES * _NUM_SUBCORES
_LANES = 16
_NBUF = 4
_OBUF = 2
_SEQ_PER_CHUNK = 2


def _positional_encoding(max_len, d_model):
    pos = jnp.arange(max_len, dtype=jnp.float32)[:, None]
    index = jnp.arange(d_model, dtype=jnp.float32)[None, :]
    pe = pos / jnp.power(10000.0, (index - index % 2) / float(d_model))
    pe_s = jnp.sin(pe[:, 0::2])[..., None]
    pe_c = jnp.cos(pe[:, 1::2])[..., None]
    return jnp.concatenate([pe_s, pe_c], axis=-1).reshape(pe.shape[0], -1)


def _interleave_perm(d):
    perm = np.empty(d, dtype=np.int32)
    for g in range(d // 32):
        for l in range(16):
            perm[g * 32 + 2 * l] = g * 32 + l
            perm[g * 32 + 2 * l + 1] = g * 32 + 16 + l
    return perm


@functools.partial(jax.jit, static_argnames=("seq_len", "d"))
def _lookup(idx, table_bf, pe_flat, seq_len, d):
    (n,) = idx.shape
    per_w = n // _NW
    ch = _SEQ_PER_CHUNK * seq_len
    n_ch = per_w // ch
    subs = []
    off = 0
    while off < ch:
        klen = min(104, ch - off)
        subs.append((off, klen))
        off += klen
    scale = float(math.sqrt(d))
    mesh = plsc.VectorSubcoreMesh(core_axis_name="c", subcore_axis_name="s")

    @functools.partial(
        pl.kernel,
        out_type=jax.ShapeDtypeStruct((n, d), jnp.float32),
        mesh=mesh,
        compiler_params=pltpu.CompilerParams(
            use_tc_tiling_on_sc=False, needs_layout_passes=False
        ),
        scratch_types=[
            pltpu.VMEM((per_w,), jnp.int32),
            pltpu.VMEM((_NBUF, ch, d), jnp.bfloat16),
            pltpu.VMEM((_OBUF, ch, d), jnp.float32),
            pltpu.VMEM((seq_len * d,), jnp.float32),
            [pltpu.SemaphoreType.DMA] * _NBUF,
            [pltpu.SemaphoreType.DMA] * _OBUF,
        ],
    )
    def k(tab_hbm, idx_hbm, pe_hbm, out_hbm,
          idx_v, rows_v, out_v, pe_v, gsems, ssems):
        wid = lax.axis_index("s") * _NUM_CORES + lax.axis_index("c")
        pltpu.sync_copy(pe_hbm, pe_v)
        pltpu.sync_copy(idx_hbm.at[pl.ds(wid * per_w, per_w)], idx_v)

        def _gather_copies(c, b):
            return [
                pltpu.make_async_copy(
                    tab_hbm.at[idx_v.at[pl.ds(c * ch + o, klen)]],
                    rows_v.at[b].at[pl.ds(o, klen)],
                    gsems[b],
                )
                for o, klen in subs
            ]

        def start_gather(c, b):
            for cp in _gather_copies(c, b):
                cp.start()

        def wait_gather(c, b):
            for cp in _gather_copies(c, b):
                cp.wait()

        def start_scatter(c, ob):
            row0 = wid * per_w + c * ch
            pltpu.async_copy(out_v.at[ob], out_hbm.at[pl.ds(row0, ch)],
                             ssems[ob])

        def wait_scatter(ob):
            pltpu.make_async_copy(
                out_v.at[ob], out_hbm.at[pl.ds(0, ch)], ssems[ob]
            ).wait()

        def compute(bb, ob):
            src = rows_v.at[bb]
            dst = out_v.at[ob]

            @plsc.parallel_loop(0, seq_len, unroll=2)
            def _(p):
                q = p + seq_len
                for g in range(d // 32):
                    x = src[p, pl.ds(g * 32, 32)]
                    y = src[q, pl.ds(g * 32, 32)]
                    xa, xb = plsc.unpack(
                        x, format=plsc.PackFormat.INTERLEAVED,
                        preferred_element_type=jnp.float32)
                    ya, yb = plsc.unpack(
                        y, format=plsc.PackFormat.INTERLEAVED,
                        preferred_element_type=jnp.float32)
                    pa = pe_v[pl.ds(p * d + g * 32, _LANES)]
                    pb = pe_v[pl.ds(p * d + g * 32 + _LANES, _LANES)]
                    dst[p, pl.ds(g * 32, _LANES)] = xa * scale + pa
                    dst[p, pl.ds(g * 32 + _LANES, _LANES)] = xb * scale + pb
                    dst[q, pl.ds(g * 32, _LANES)] = ya * scale + pa
                    dst[q, pl.ds(g * 32 + _LANES, _LANES)] = yb * scale + pb

        start_gather(0, 0)
        start_gather(1, 1)

        def outer(o, carry):
            for bb in range(_NBUF):
                c = o * _NBUF + bb
                ob = bb % _OBUF
                bn = (bb + 2) % _NBUF

                @pl.when(c + 2 < n_ch)
                def _():
                    start_gather(c + 2, bn)

                wait_gather(c, bb)

                @pl.when(c >= _OBUF)
                def _():
                    wait_scatter(ob)

                compute(bb, ob)
                start_scatter(c, ob)
            return carry

        lax.fori_loop(0, n_ch // _NBUF, outer, 0)
        wait_scatter(0)
        wait_scatter(1)

    return k(table_bf, idx, pe_flat)


def kernel(sequences, table):
    b, s = sequences.shape
    v, d = table.shape
    n = b * s
    idx = sequences.astype(jnp.int32).reshape(n)
    perm = jnp.asarray(_interleave_perm(d))
    table_bf = table.astype(jnp.bfloat16)[:, perm]
    pe_flat = _positional_encoding(s, d).reshape(s * d)
    out = _lookup(idx, table_bf, pe_flat, s, d)
    return out.reshape(b, s, d)


# speedup vs baseline: 2.0703x; 2.0703x over previous
import functools
import math

import jax
import jax.numpy as jnp
from jax import lax
from jax.experimental import pallas as pl
from jax.experimental.pallas import tpu as pltpu
from jax.experimental.pallas import tpu_sc as plsc

_NUM_CORES = 2
_NUM_SUBCORES = 16
_NW = _NUM_CORES * _NUM_SUBCORES
_LANES = 16
_NBUF = 4
_SEQ_PER_CHUNK = 2


def _positional_encoding(max_len, d_model):
    pos = jnp.arange(max_len, dtype=jnp.float32)[:, None]
    index = jnp.arange(d_model, dtype=jnp.float32)[None, :]
    pe = pos / jnp.power(10000.0, (index - index % 2) / float(d_model))
    pe_s = jnp.sin(pe[:, 0::2])[..., None]
    pe_c = jnp.cos(pe[:, 1::2])[..., None]
    return jnp.concatenate([pe_s, pe_c], axis=-1).reshape(pe.shape[0], -1)


@functools.partial(jax.jit, static_argnames=("v", "seq_len", "d"))
def _lookup(idx, table_flat, pe_flat, v, seq_len, d):
    (n,) = idx.shape
    per_w = n // _NW
    ch = _SEQ_PER_CHUNK * seq_len
    n_ch = per_w // ch
    subs = []
    off = 0
    while off < ch:
        klen = min(104, ch - off)
        subs.append((off, klen))
        off += klen
    scale = float(math.sqrt(d))
    mesh = plsc.VectorSubcoreMesh(core_axis_name="c", subcore_axis_name="s")

    @functools.partial(
        pl.kernel,
        out_type=jax.ShapeDtypeStruct((n, d), jnp.float32),
        mesh=mesh,
        compiler_params=pltpu.CompilerParams(use_tc_tiling_on_sc=False),
        scratch_types=[
            pltpu.VMEM((per_w,), jnp.int32),
            pltpu.VMEM((_NBUF, ch, d), jnp.float32),
            pltpu.VMEM((seq_len * d,), jnp.float32),
            [pltpu.SemaphoreType.DMA] * _NBUF,
            [pltpu.SemaphoreType.DMA] * _NBUF,
        ],
    )
    def k(tab_hbm, idx_hbm, pe_hbm, out_hbm,
          idx_v, rows_v, pe_v, gsems, ssems):
        wid = lax.axis_index("s") * _NUM_CORES + lax.axis_index("c")
        tab2d = tab_hbm
        out2d = out_hbm
        pltpu.sync_copy(pe_hbm, pe_v)
        pltpu.sync_copy(idx_hbm.at[pl.ds(wid * per_w, per_w)], idx_v)

        def _gather_copies(c, b):
            return [
                pltpu.make_async_copy(
                    tab2d.at[idx_v.at[pl.ds(c * ch + o, klen)]],
                    rows_v.at[b].at[pl.ds(o, klen)],
                    gsems[b],
                )
                for o, klen in subs
            ]

        def start_gather(c, b):
            for cp in _gather_copies(c, b):
                cp.start()

        def wait_gather(c, b):
            for cp in _gather_copies(c, b):
                cp.wait()

        def start_scatter(c, b):
            row0 = wid * per_w + c * ch
            pltpu.async_copy(rows_v.at[b], out2d.at[pl.ds(row0, ch)], ssems[b])

        def wait_scatter(b):
            pltpu.make_async_copy(
                rows_v.at[b], out2d.at[pl.ds(0, ch)], ssems[b]
            ).wait()

        def compute(b):
            buf = rows_v.at[b]

            @plsc.parallel_loop(0, seq_len, unroll=2)
            def _(p):
                for t in range(d // _LANES):
                    sl = pl.ds(t * _LANES, _LANES)
                    pe_val = pe_v[pl.ds(p * d + t * _LANES, _LANES)]
                    buf[p, sl] = buf[p, sl] * scale + pe_val
                    q = p + seq_len
                    buf[q, sl] = buf[q, sl] * scale + pe_val

        start_gather(0, 0)
        start_gather(1, 1)

        def outer(o, carry):
            for bb in range(_NBUF):
                c = o * _NBUF + bb
                bn = (bb + 2) % _NBUF

                @pl.when(c + 2 < n_ch)
                def _():
                    @pl.when(c >= 2)
                    def _():
                        wait_scatter(bn)

                    start_gather(c + 2, bn)

                wait_gather(c, bb)
                compute(bb)
                start_scatter(c, bb)
            return carry

        lax.fori_loop(0, n_ch // _NBUF, outer, 0)
        wait_scatter((n_ch - 2) % _NBUF)
        wait_scatter((n_ch - 1) % _NBUF)

    return k(table_flat.reshape(v, d), idx, pe_flat)


def kernel(sequences, table):
    b, s = sequences.shape
    v, d = table.shape
    n = b * s
    idx = sequences.astype(jnp.int32).reshape(n)
    table_flat = table.reshape(v * d)
    pe_flat = _positional_encoding(s, d).reshape(s * d)
    out = _lookup(idx, table_flat, pe_flat, v, s, d)
    return out.reshape(b, s, d)


# speedup vs baseline: 2.0809x; 1.0051x over previous
import functools
import math

import jax
import jax.numpy as jnp
from jax import lax
from jax.experimental import pallas as pl
from jax.experimental.pallas import tpu as pltpu
from jax.experimental.pallas import tpu_sc as plsc

_NUM_CORES = 2
_NUM_SUBCORES = 16
_NW = _NUM_CORES * _NUM_SUBCORES
_LANES = 16
_NBUF = 4
_SEQ_PER_CHUNK = 2


def _positional_encoding(max_len, d_model):
    pos = jnp.arange(max_len, dtype=jnp.float32)[:, None]
    index = jnp.arange(d_model, dtype=jnp.float32)[None, :]
    pe = pos / jnp.power(10000.0, (index - index % 2) / float(d_model))
    pe_s = jnp.sin(pe[:, 0::2])[..., None]
    pe_c = jnp.cos(pe[:, 1::2])[..., None]
    return jnp.concatenate([pe_s, pe_c], axis=-1).reshape(pe.shape[0], -1)


@functools.partial(jax.jit, static_argnames=("v", "seq_len", "d"))
def _lookup(idx, table_flat, pe_flat, v, seq_len, d):
    (n,) = idx.shape
    per_w = n // _NW
    ch = _SEQ_PER_CHUNK * seq_len
    n_ch = per_w // ch
    subs = []
    off = 0
    while off < ch:
        klen = min(104, ch - off)
        subs.append((off, klen))
        off += klen
    scale = float(math.sqrt(d))
    mesh = plsc.VectorSubcoreMesh(core_axis_name="c", subcore_axis_name="s")

    @functools.partial(
        pl.kernel,
        out_type=jax.ShapeDtypeStruct((n, d), jnp.float32),
        mesh=mesh,
        compiler_params=pltpu.CompilerParams(use_tc_tiling_on_sc=False),
        scratch_types=[
            pltpu.VMEM((per_w,), jnp.int32),
            pltpu.VMEM((_NBUF, ch, d), jnp.float32),
            pltpu.VMEM((seq_len * d,), jnp.float32),
            [pltpu.SemaphoreType.DMA] * _NBUF,
            [pltpu.SemaphoreType.DMA] * _NBUF,
        ],
    )
    def k(tab_hbm, idx_hbm, pe_hbm, out_hbm,
          idx_v, rows_v, pe_v, gsems, ssems):
        wid = lax.axis_index("s") * _NUM_CORES + lax.axis_index("c")
        tab2d = tab_hbm
        out2d = out_hbm
        pltpu.sync_copy(pe_hbm, pe_v)
        pltpu.sync_copy(idx_hbm.at[pl.ds(wid * per_w, per_w)], idx_v)

        def _gather_copies(c, b):
            return [
                pltpu.make_async_copy(
                    tab2d.at[idx_v.at[pl.ds(c * ch + o, klen)]],
                    rows_v.at[b].at[pl.ds(o, klen)],
                    gsems[b],
                )
                for o, klen in subs
            ]

        def start_gather(c, b):
            for cp in _gather_copies(c, b):
                cp.start()

        def wait_gather(c, b):
            for cp in _gather_copies(c, b):
                cp.wait()

        def start_scatter(c, b):
            row0 = wid * per_w + c * ch
            pltpu.async_copy(rows_v.at[b], out2d.at[pl.ds(row0, ch)], ssems[b])

        def wait_scatter(b):
            pltpu.make_async_copy(
                rows_v.at[b], out2d.at[pl.ds(0, ch)], ssems[b]
            ).wait()

        def compute(b):
            buf = rows_v.at[b]

            @plsc.parallel_loop(0, seq_len, unroll=4)
            def _(p):
                for t in range(d // _LANES):
                    sl = pl.ds(t * _LANES, _LANES)
                    pe_val = pe_v[pl.ds(p * d + t * _LANES, _LANES)]
                    buf[p, sl] = buf[p, sl] * scale + pe_val
                    q = p + seq_len
                    buf[q, sl] = buf[q, sl] * scale + pe_val

        start_gather(0, 0)
        start_gather(1, 1)

        def outer(o, carry):
            for bb in range(_NBUF):
                c = o * _NBUF + bb
                bn = (bb + 2) % _NBUF

                @pl.when(c + 2 < n_ch)
                def _():
                    @pl.when(c >= 2)
                    def _():
                        wait_scatter(bn)

                    start_gather(c + 2, bn)

                wait_gather(c, bb)
                compute(bb)
                start_scatter(c, bb)
            return carry

        lax.fori_loop(0, n_ch // _NBUF, outer, 0)
        wait_scatter((n_ch - 2) % _NBUF)
        wait_scatter((n_ch - 1) % _NBUF)

    return k(table_flat.reshape(v, d), idx, pe_flat)


def kernel(sequences, table):
    b, s = sequences.shape
    v, d = table.shape
    n = b * s
    idx = sequences.astype(jnp.int32).reshape(n)
    table_flat = table.reshape(v * d)
    pe_flat = _positional_encoding(s, d).reshape(s * d)
    out = _lookup(idx, table_flat, pe_flat, v, s, d)
    return out.reshape(b, s, d)
